# baseline (speedup 1.0000x reference)
import math
import numpy as np
import jax
import jax.numpy as jnp
from jax.experimental import pallas as pl
from jax.experimental.pallas import tpu as pltpu

_G = math.exp(-1.0 / (2.0 * 1.5 * 1.5))
_K0 = _G / (1.0 + 2.0 * _G)
_K1 = 1.0 / (1.0 + 2.0 * _G)


def _bilinear_matrix(in_size: int, scale: int) -> np.ndarray:
    out_size = in_size * scale
    o = np.arange(out_size, dtype=np.float64)
    src = np.clip((o + 0.5) * (in_size / out_size) - 0.5, 0.0, None)
    i0 = np.minimum(np.floor(src).astype(np.int64), in_size - 1)
    i1 = np.minimum(i0 + 1, in_size - 1)
    wgt = src - i0
    m = np.zeros((out_size, in_size), dtype=np.float64)
    m[np.arange(out_size), i0] += 1.0 - wgt
    m[np.arange(out_size), i1] += wgt
    return m


def _blur_band_matrix(n: int) -> np.ndarray:
    g = np.zeros((n, n), dtype=np.float64)
    for i in range(n):
        for off, kk in ((-1, _K0), (0, _K1), (1, _K0)):
            j = i + off
            if j < 0:
                j = -j
            elif j > n - 1:
                j = 2 * (n - 1) - j
            g[i, j] += kk
    return g


def _make_body(bch: int, pk: int, sh: int, sw: int):
    nq = bch // pk

    def _body(x_ref, r_ref, a_ref, o_ref):
        h = x_ref.shape[1]
        w = x_ref.shape[2]
        xb = x_ref[...].reshape(bch * h, w)
        t = jnp.dot(xb, r_ref[...], preferred_element_type=jnp.float32)
        t = t.astype(jnp.bfloat16).reshape(nq, pk * h, sw)
        a = a_ref[...]
        for q in range(nq):
            y = jnp.dot(a, t[q], preferred_element_type=jnp.float32)
            o_ref[q * pk:(q + 1) * pk] = y.reshape(pk, sh, sw)

    return _body


def _make_stream_body(bch: int, pk: int, sh: int, sw: int, g: int):
    nq = bch // pk

    def _body(x_ref, r_ref, a_ref, o_hbm, ybuf, sems):
        h = x_ref.shape[1]
        w = x_ref.shape[2]
        r = r_ref[...]
        a = a_ref[...]
        for i in range(g):
            slot = i % 2
            if i >= 2:
                pltpu.make_async_copy(ybuf.at[slot], ybuf.at[slot],
                                      sems.at[slot]).wait()
            xb = x_ref[i * bch:(i + 1) * bch].reshape(bch * h, w)
            t = jnp.dot(xb, r, preferred_element_type=jnp.float32)
            t = t.astype(jnp.bfloat16).reshape(nq, pk * h, sw)
            for q in range(nq):
                y = jnp.dot(a, t[q], preferred_element_type=jnp.float32)
                ybuf[slot, q * pk:(q + 1) * pk] = y.reshape(pk, sh, sw)
            pltpu.make_async_copy(ybuf.at[slot],
                                  o_hbm.at[pl.ds(i * bch, bch)],
                                  sems.at[slot]).start()
        for slot in range(min(2, g)):
            pltpu.make_async_copy(ybuf.at[slot], ybuf.at[slot],
                                  sems.at[slot]).wait()

    return _body


def _blur_upsample_planes(xp: jax.Array, s: int) -> jax.Array:
    m, h, w = xp.shape
    sh, sw = s * h, s * w

    a_np = _bilinear_matrix(h, s) @ _blur_band_matrix(h)
    r_np = (_bilinear_matrix(w, s) @ _blur_band_matrix(w)).T

    pk = 1
    for cand in (4, 2):
        if m % cand == 0 and cand * h <= 256:
            pk = cand
            break
    a_bd = np.zeros((pk * sh, pk * h), dtype=np.float64)
    for b in range(pk):
        a_bd[b * sh:(b + 1) * sh, b * h:(b + 1) * h] = a_np
    a_bd = jnp.asarray(a_bd, dtype=jnp.bfloat16)
    r_bf = jnp.asarray(r_np, dtype=jnp.bfloat16)

    bch = pk
    for d in range(m, 0, -1):
        if m % d == 0 and d % pk == 0 and d * (h * w * 2 + sh * sw * 4) <= (16 << 20):
            if m // d >= 8 or d == m:
                bch = d
                break
    g = m // bch

    flops = m * (2 * sh * h * w + 2 * sh * w * sw)
    bytes_accessed = int(xp.size * 2 + m * sh * sw * 4 + a_bd.size * 2
                         + r_bf.size * 2)

    if m * h * w * 2 <= (12 << 20) and g >= 2:
        return pl.pallas_call(
            _make_stream_body(bch, pk, sh, sw, g),
            out_shape=jax.ShapeDtypeStruct((m, sh, sw), jnp.float32),
            in_specs=[
                pl.BlockSpec((m, h, w), lambda: (0, 0, 0)),
                pl.BlockSpec((w, sw), lambda: (0, 0)),
                pl.BlockSpec((pk * sh, pk * h), lambda: (0, 0)),
            ],
            out_specs=pl.BlockSpec(memory_space=pl.MemorySpace.ANY),
            scratch_shapes=[
                pltpu.VMEM((2, bch, sh, sw), jnp.float32),
                pltpu.SemaphoreType.DMA((2,)),
            ],
            cost_estimate=pl.CostEstimate(flops=int(flops), transcendentals=0,
                                          bytes_accessed=bytes_accessed),
        )(xp, r_bf, a_bd)

    return pl.pallas_call(
        _make_body(bch, pk, sh, sw),
        out_shape=jax.ShapeDtypeStruct((m, sh, sw), jnp.float32),
        grid=(g,),
        in_specs=[
            pl.BlockSpec((bch, h, w), lambda i: (i, 0, 0)),
            pl.BlockSpec((w, sw), lambda i: (0, 0),
                         pipeline_mode=pl.Buffered(1)),
            pl.BlockSpec((pk * sh, pk * h), lambda i: (0, 0),
                         pipeline_mode=pl.Buffered(1)),
        ],
        out_specs=pl.BlockSpec((bch, sh, sw), lambda i: (i, 0, 0)),
        compiler_params=pltpu.CompilerParams(
            dimension_semantics=("arbitrary",)),
        cost_estimate=pl.CostEstimate(flops=int(flops), transcendentals=0,
                                      bytes_accessed=bytes_accessed),
    )(xp, r_bf, a_bd)


def kernel(x):
    n, c, h, w = x.shape
    s = 2
    out = _blur_upsample_planes(x.reshape(n * c, h, w).astype(jnp.bfloat16), s)
    return out.reshape(n, c, s * h, s * w)


# speedup vs baseline: 1.0267x; 1.0267x over previous
import math
import numpy as np
import jax
import jax.numpy as jnp
from jax.experimental import pallas as pl
from jax.experimental.pallas import tpu as pltpu

_G = math.exp(-1.0 / (2.0 * 1.5 * 1.5))
_K0 = _G / (1.0 + 2.0 * _G)
_K1 = 1.0 / (1.0 + 2.0 * _G)


def _bilinear_matrix(in_size: int, scale: int) -> np.ndarray:
    out_size = in_size * scale
    o = np.arange(out_size, dtype=np.float64)
    src = np.clip((o + 0.5) * (in_size / out_size) - 0.5, 0.0, None)
    i0 = np.minimum(np.floor(src).astype(np.int64), in_size - 1)
    i1 = np.minimum(i0 + 1, in_size - 1)
    wgt = src - i0
    m = np.zeros((out_size, in_size), dtype=np.float64)
    m[np.arange(out_size), i0] += 1.0 - wgt
    m[np.arange(out_size), i1] += wgt
    return m


def _blur_band_matrix(n: int) -> np.ndarray:
    g = np.zeros((n, n), dtype=np.float64)
    for i in range(n):
        for off, kk in ((-1, _K0), (0, _K1), (1, _K0)):
            j = i + off
            if j < 0:
                j = -j
            elif j > n - 1:
                j = 2 * (n - 1) - j
            g[i, j] += kk
    return g


def _make_body(bch: int, pk: int, sh: int, sw: int):
    nq = bch // pk

    def _body(x_ref, r_ref, a_ref, o_ref):
        h = x_ref.shape[1]
        w = x_ref.shape[2]
        xb = x_ref[...].reshape(bch * h, w)
        t = jnp.dot(xb, r_ref[...], preferred_element_type=jnp.float32)
        t = t.astype(jnp.bfloat16).reshape(nq, pk * h, sw)
        a = a_ref[...]
        for q in range(nq):
            y = jnp.dot(a, t[q], preferred_element_type=jnp.float32)
            o_ref[q * pk:(q + 1) * pk] = y.reshape(pk, sh, sw)

    return _body


def _blur_upsample_planes(xp: jax.Array, s: int) -> jax.Array:
    m, h, w = xp.shape
    sh, sw = s * h, s * w

    a_np = _bilinear_matrix(h, s) @ _blur_band_matrix(h)
    r_np = (_bilinear_matrix(w, s) @ _blur_band_matrix(w)).T

    pk = 1
    for cand in (4, 2):
        if m % cand == 0 and cand * h <= 256:
            pk = cand
            break
    a_bd = np.zeros((pk * sh, pk * h), dtype=np.float64)
    for b in range(pk):
        a_bd[b * sh:(b + 1) * sh, b * h:(b + 1) * h] = a_np
    a_bd = jnp.asarray(a_bd, dtype=jnp.bfloat16)
    r_bf = jnp.asarray(r_np, dtype=jnp.bfloat16)

    bch = pk
    for d in range(m, 0, -1):
        if m % d == 0 and d % pk == 0 and d * (h * w * 2 + sh * sw * 4) <= (16 << 20):
            if m // d >= 8 or d == m:
                bch = d
                break
    g = m // bch

    flops = m * (2 * sh * h * w + 2 * sh * w * sw)
    bytes_accessed = int(xp.size * 2 + m * sh * sw * 4 + a_bd.size * 2
                         + r_bf.size * 2)

    return pl.pallas_call(
        _make_body(bch, pk, sh, sw),
        out_shape=jax.ShapeDtypeStruct((m, sh, sw), jnp.float32),
        grid=(g,),
        in_specs=[
            pl.BlockSpec((bch, h, w), lambda i: (i, 0, 0)),
            pl.BlockSpec((w, sw), lambda i: (0, 0),
                         pipeline_mode=pl.Buffered(1)),
            pl.BlockSpec((pk * sh, pk * h), lambda i: (0, 0),
                         pipeline_mode=pl.Buffered(1)),
        ],
        out_specs=pl.BlockSpec((bch, sh, sw), lambda i: (i, 0, 0)),
        compiler_params=pltpu.CompilerParams(
            dimension_semantics=("arbitrary",)),
        cost_estimate=pl.CostEstimate(flops=int(flops), transcendentals=0,
                                      bytes_accessed=bytes_accessed),
    )(xp, r_bf, a_bd)


def kernel(x):
    n, c, h, w = x.shape
    s = 2
    out = _blur_upsample_planes(x.reshape(n * c, h, w).astype(jnp.bfloat16), s)
    return out.reshape(n, c, s * h, s * w)
